# baseline (speedup 1.0000x reference)
import functools

import jax
import jax.numpy as jnp
from jax import lax
from jax.experimental import pallas as pl
from jax.experimental.pallas import tpu as pltpu
from jax.experimental.pallas import tpu_sc as plsc

_D = 64
_B = 16384
_CW = 256
_CSH = 8
_MAIN = 999936
_NCH = _MAIN // _CW
_TAILC = _NCH


def _popcnt(mask):
    return plsc.all_reduce_population_count(mask)[0]


def _compress_store(ref, start, x, mask):
    pos = start + plsc.cumsum(jnp.where(mask, 1, 0)) - 1
    pos = jnp.where(mask, pos, 0)
    plsc.store_scatter(ref, [pos], x, mask=mask)
    return _popcnt(mask)


def _build(num_cores, num_subcores):
    nw = num_cores * num_subcores
    n_kk = -4 * (-(_NCH // nw + 1) // 4)
    mesh = plsc.VectorSubcoreMesh(core_axis_name="c", subcore_axis_name="s")

    @functools.partial(
        pl.kernel,
        mesh=mesh,
        out_type=jax.ShapeDtypeStruct((_B, 1, _D), jnp.float32),
        scratch_types=[
            pltpu.VMEM((_B,), jnp.int32),
            pltpu.VMEM((_B + 16,), jnp.int32),
            pltpu.VMEM((_B + 16,), jnp.int32),
            pltpu.VMEM((_D, _CW), jnp.float32),
            pltpu.VMEM((_D, _CW), jnp.float32),
            pltpu.VMEM((_D, _CW), jnp.float32),
            pltpu.VMEM((_D, _CW), jnp.float32),
            pltpu.VMEM((_D, _D), jnp.float32),
            pltpu.VMEM((16, 1, _D), jnp.float32),
            pltpu.SemaphoreType.DMA,
            pltpu.SemaphoreType.DMA,
            pltpu.SemaphoreType.DMA,
            pltpu.SemaphoreType.DMA,
            pltpu.SemaphoreType.DMA,
        ],
        compiler_params=pltpu.CompilerParams(needs_layout_passes=False),
    )
    def k(idx_hbm, table_hbm, tail_hbm, out_hbm,
          idx_v, me_j, cl, buf_a, buf_b, buf_c, buf_d, tailbuf, ring,
          sem_a, sem_b, sem_c, sem_d, osem):
        wid = lax.axis_index("s") * num_cores + lax.axis_index("c")
        nmine = jnp.where(wid < _NCH % nw, _NCH // nw + 1, _NCH // nw)
        iota16 = lax.iota(jnp.int32, 16)
        bufs = (buf_a, buf_b, buf_c, buf_d)
        sems = (sem_a, sem_b, sem_c, sem_d)

        def issue(kk, buf, sem):
            c = wid + kk * nw
            c_dma = jnp.where(kk < nmine, c, 0)
            pltpu.async_copy(
                table_hbm.at[:, pl.ds(c_dma * _CW, _CW)], buf, sem)

        for r in range(4):
            issue(jnp.int32(r), bufs[r], sems[r])

        for s in range(16):
            pltpu.async_copy(
                out_hbm.at[pl.ds(0, 1)], ring.at[pl.ds(s, 1)], osem)

        pltpu.sync_copy(idx_hbm, idx_v)
        pltpu.sync_copy(tail_hbm, tailbuf)

        def collect(g, cur):
            ev = idx_v[pl.ds(g * 16, 16)]
            jv = iota16 + g * 16
            own = ((ev >> _CSH) & (nw - 1)) == wid
            return cur + _compress_store(me_j, cur, jv, own)

        n_me = pl.loop(0, _B // 16, init_carry=jnp.int32(0))(collect)
        n_me_g = (n_me + 15) >> 4

        def chunk_pairs(c):
            def scan(g, cc):
                valid = (iota16 + g * 16) < n_me
                jv = jnp.where(valid, me_j[pl.ds(g * 16, 16)], 0)
                ev = plsc.load_gather(idx_v, [jv], mask=valid)
                m = ((ev >> _CSH) == c) & valid
                packed = ((ev & (_CW - 1)) << 14) | jv
                return cc + _compress_store(cl, cc, packed, m)

            return pl.loop(0, n_me_g, init_carry=jnp.int32(0))(scan)

        def emit_matches(n_pairs, ocnt0, load):
            def one(i, ocnt):
                pk = plsc.load_gather(cl, [jnp.full((16,), i, jnp.int32)])
                colv = pk >> 14
                j = pk[0] & (_B - 1)
                slot = ocnt & 15
                pltpu.make_async_copy(
                    out_hbm.at[pl.ds(0, 1)], ring.at[pl.ds(0, 1)], osem
                ).wait()
                for t in range(_D // 16):
                    ring[slot, 0, pl.ds(t * 16, 16)] = load(t, colv)
                pltpu.async_copy(
                    ring.at[pl.ds(slot, 1)], out_hbm.at[pl.ds(j, 1)], osem)
                return ocnt + 1

            return pl.loop(0, n_pairs, init_carry=ocnt0)(one)

        def process(kk, buf, sem, ocnt):
            pltpu.make_async_copy(
                table_hbm.at[:, pl.ds(0, _CW)], buf, sem).wait()
            c = jnp.where(kk < nmine, wid + kk * nw, -1)
            n_pairs = chunk_pairs(c)

            def load(t, colv):
                rows = iota16 + t * 16
                return plsc.load_gather(buf, [rows, colv])

            return emit_matches(n_pairs, ocnt, load)

        def body(q, ocnt):
            for r in range(4):
                kk = 4 * q + r
                ocnt = process(kk, bufs[r], sems[r], ocnt)
                issue(kk + 4, bufs[r], sems[r])
            return ocnt

        ocnt = pl.loop(0, n_kk // 4, init_carry=jnp.int32(0))(body)
        for r in range(4):
            pltpu.make_async_copy(
                table_hbm.at[:, pl.ds(0, _CW)], bufs[r], sems[r]).wait()

        n_tail = chunk_pairs(jnp.int32(_TAILC))

        def tail_load(t, colv):
            rows = iota16 + t * 16
            return plsc.load_gather(tailbuf, [rows, colv])

        ocnt = emit_matches(n_tail, ocnt, tail_load)

        del ocnt
        for _ in range(16):
            pltpu.make_async_copy(
                out_hbm.at[pl.ds(0, 1)], ring.at[pl.ds(0, 1)], osem).wait()

    return k


def kernel(entities, entity_embeddings):
    info = plsc.get_sparse_core_info()
    fn = _build(info.num_cores, info.num_subcores)
    tail = entity_embeddings[_MAIN:].T
    out = fn(entities.astype(jnp.int32), entity_embeddings.T, tail)
    return out.reshape(_B, _D)


# speedup vs baseline: 1.0593x; 1.0593x over previous
import functools

import jax
import jax.numpy as jnp
from jax import lax
from jax.experimental import pallas as pl
from jax.experimental.pallas import tpu as pltpu
from jax.experimental.pallas import tpu_sc as plsc

_D = 64
_B = 16384
_CW = 256
_CSH = 8
_MAIN = 999936
_NCH = _MAIN // _CW


def _popcnt(mask):
    return plsc.all_reduce_population_count(mask)[0]


def _compress_store(ref, start, x, mask):
    pos = start + plsc.cumsum(jnp.where(mask, 1, 0)) - 1
    pos = jnp.where(mask, pos, 0)
    plsc.store_scatter(ref, [pos], x, mask=mask)
    return _popcnt(mask)


def _build(num_cores, num_subcores):
    nw = num_cores * num_subcores
    n_mk = ((_NCH // nw + 2) // 2 + 1) // 2 * 2
    mesh = plsc.VectorSubcoreMesh(core_axis_name="c", subcore_axis_name="s")

    @functools.partial(
        pl.kernel,
        mesh=mesh,
        out_type=jax.ShapeDtypeStruct((_B, 1, _D), jnp.float32),
        scratch_types=[
            pltpu.VMEM((_B,), jnp.int32),
            pltpu.VMEM((_B + 16,), jnp.int32),
            pltpu.VMEM((_B + 16,), jnp.int32),
            pltpu.VMEM((4, _D, _CW), jnp.float32),
            pltpu.VMEM((_D, _D), jnp.float32),
            pltpu.VMEM((16, 1, _D), jnp.float32),
            pltpu.SemaphoreType.DMA,
            pltpu.SemaphoreType.DMA,
            pltpu.SemaphoreType.DMA,
        ],
        compiler_params=pltpu.CompilerParams(needs_layout_passes=False),
    )
    def k(idx_hbm, table_hbm, tail_hbm, out_hbm,
          idx_v, me_j, cl, buf4, tailbuf, ring,
          sem_a, sem_b, osem):
        wid = lax.axis_index("s") * num_cores + lax.axis_index("c")
        nmine = jnp.where(wid < _NCH % nw, _NCH // nw + 1, _NCH // nw)
        iota16 = lax.iota(jnp.int32, 16)
        sems = (sem_a, sem_b)

        def issue_macro(m, par, sem):
            for r in range(2):
                kk = 2 * m + r
                c_dma = jnp.where(kk < nmine, wid + kk * nw, 0)
                pltpu.async_copy(
                    table_hbm.at[:, pl.ds(c_dma * _CW, _CW)],
                    buf4.at[par * 2 + r], sem)

        issue_macro(jnp.int32(0), 0, sem_a)
        issue_macro(jnp.int32(1), 1, sem_b)

        for s in range(16):
            pltpu.async_copy(
                out_hbm.at[pl.ds(0, 1)], ring.at[pl.ds(s, 1)], osem)

        pltpu.sync_copy(idx_hbm, idx_v)
        pltpu.sync_copy(tail_hbm, tailbuf)

        def collect(g, cur):
            ev = idx_v[pl.ds(g * 16, 16)]
            jv = iota16 + g * 16
            own = ((ev >> _CSH) & (nw - 1)) == wid
            return cur + _compress_store(me_j, cur, jv, own)

        n_me = pl.loop(0, _B // 16, init_carry=jnp.int32(0))(collect)
        n_me_g = (n_me + 15) >> 4

        def macro_pairs(m):
            def scan(g, cc):
                valid = (iota16 + g * 16) < n_me
                jv = jnp.where(valid, me_j[pl.ds(g * 16, 16)], 0)
                ev = plsc.load_gather(idx_v, [jv], mask=valid)
                mm = ((ev >> (_CSH + 6)) == m) & (ev < _MAIN) & valid
                packed = (((ev >> (_CSH + 5)) & 1) << 22) | \
                    ((ev & (_CW - 1)) << 14) | jv
                return cc + _compress_store(cl, cc, packed, mm)

            return pl.loop(0, n_me_g, init_carry=jnp.int32(0))(scan)

        def tail_pairs():
            def scan(g, cc):
                valid = (iota16 + g * 16) < n_me
                jv = jnp.where(valid, me_j[pl.ds(g * 16, 16)], 0)
                ev = plsc.load_gather(idx_v, [jv], mask=valid)
                mm = (ev >= _MAIN) & valid
                packed = ((ev & (_CW - 1)) << 14) | jv
                return cc + _compress_store(cl, cc, packed, mm)

            return pl.loop(0, n_me_g, init_carry=jnp.int32(0))(scan)

        def emit_matches(n_pairs, ocnt0, load):
            def one(i, ocnt):
                pk = plsc.load_gather(cl, [jnp.full((16,), i, jnp.int32)])
                colv = (pk >> 14) & (_CW - 1)
                sbit = pk >> 22
                j = pk[0] & (_B - 1)
                slot = ocnt & 15
                pltpu.make_async_copy(
                    out_hbm.at[pl.ds(0, 1)], ring.at[pl.ds(0, 1)], osem
                ).wait()
                for t in range(_D // 16):
                    ring[slot, 0, pl.ds(t * 16, 16)] = load(t, colv, sbit)
                pltpu.async_copy(
                    ring.at[pl.ds(slot, 1)], out_hbm.at[pl.ds(j, 1)], osem)
                return ocnt + 1

            return pl.loop(0, n_pairs, init_carry=ocnt0)(one)

        def process_macro(m, par, sem, ocnt):
            for _ in range(2):
                pltpu.make_async_copy(
                    table_hbm.at[:, pl.ds(0, _CW)],
                    buf4.at[0], sem).wait()
            n_pairs = macro_pairs(m)
            base = jnp.int32(par * 2)

            def load(t, colv, sbit):
                rows = iota16 + t * 16
                return plsc.load_gather(buf4, [base + sbit, rows, colv])

            return emit_matches(n_pairs, ocnt, load)

        def body(q, ocnt):
            m0 = 2 * q
            ocnt = process_macro(m0, 0, sem_a, ocnt)
            issue_macro(m0 + 2, 0, sem_a)
            ocnt = process_macro(m0 + 1, 1, sem_b, ocnt)
            issue_macro(m0 + 3, 1, sem_b)
            return ocnt

        ocnt = pl.loop(0, n_mk // 2, init_carry=jnp.int32(0))(body)
        for sem in sems:
            for _ in range(2):
                pltpu.make_async_copy(
                    table_hbm.at[:, pl.ds(0, _CW)],
                    buf4.at[0], sem).wait()

        n_tail = tail_pairs()

        def tail_load(t, colv, sbit):
            rows = iota16 + t * 16
            return plsc.load_gather(tailbuf, [rows, colv])

        ocnt = emit_matches(n_tail, ocnt, tail_load)

        del ocnt
        for _ in range(16):
            pltpu.make_async_copy(
                out_hbm.at[pl.ds(0, 1)], ring.at[pl.ds(0, 1)], osem).wait()

    return k


def kernel(entities, entity_embeddings):
    info = plsc.get_sparse_core_info()
    fn = _build(info.num_cores, info.num_subcores)
    tail = entity_embeddings[_MAIN:].T
    out = fn(entities.astype(jnp.int32), entity_embeddings.T, tail)
    return out.reshape(_B, _D)


# speedup vs baseline: 1.1273x; 1.0643x over previous
import functools

import jax
import jax.numpy as jnp
from jax import lax
from jax.experimental import pallas as pl
from jax.experimental.pallas import tpu as pltpu
from jax.experimental.pallas import tpu_sc as plsc

_D = 64
_B = 16384
_CW = 256
_MAIN = 999936
_NCH = _MAIN // _CW
_TAILC = _NCH


def _popcnt(mask):
    return plsc.all_reduce_population_count(mask)[0]


def _compress_store(ref, start, x, mask):
    pos = start + plsc.cumsum(jnp.where(mask, 1, 0)) - 1
    pos = jnp.where(mask, pos, 0)
    plsc.store_scatter(ref, [pos], x, mask=mask)
    return _popcnt(mask)


def _build(num_cores, num_subcores):
    nw = num_cores * num_subcores
    n_kk = -3 * (-(_NCH // nw + 1) // 3)
    mesh = plsc.VectorSubcoreMesh(core_axis_name="c", subcore_axis_name="s")

    @functools.partial(
        pl.kernel,
        mesh=mesh,
        out_type=jax.ShapeDtypeStruct((_B, 1, _D), jnp.float32),
        scratch_types=[
            pltpu.VMEM((_B,), jnp.int32),
            pltpu.VMEM((_B + 16,), jnp.int32),
            pltpu.VMEM((_B + 16,), jnp.int32),
            pltpu.VMEM((_B + 16,), jnp.int32),
            pltpu.VMEM((_D, _CW), jnp.float32),
            pltpu.VMEM((_D, _CW), jnp.float32),
            pltpu.VMEM((_D, _CW), jnp.float32),
            pltpu.VMEM((_D, _D), jnp.float32),
            pltpu.VMEM((16, 1, _D), jnp.float32),
            pltpu.SemaphoreType.DMA,
            pltpu.SemaphoreType.DMA,
            pltpu.SemaphoreType.DMA,
            pltpu.SemaphoreType.DMA,
        ],
        compiler_params=pltpu.CompilerParams(needs_layout_passes=False),
    )
    def k(idx_hbm, table_hbm, tail_hbm, out_hbm,
          idx_v, me_e, me_j, cl, buf_a, buf_b, buf_c, tailbuf, ring,
          sem_a, sem_b, sem_c, osem):
        wid = lax.axis_index("s") * num_cores + lax.axis_index("c")
        nmine = jnp.where(wid < _NCH % nw, _NCH // nw + 1, _NCH // nw)
        iota16 = lax.iota(jnp.int32, 16)
        bufs = (buf_a, buf_b, buf_c)
        sems = (sem_a, sem_b, sem_c)

        def issue(kk, buf, sem):
            c = wid + kk * nw
            c_dma = jnp.where(kk < nmine, c, 0)
            pltpu.async_copy(
                table_hbm.at[:, pl.ds(c_dma * _CW, _CW)], buf, sem)

        for r in range(3):
            issue(jnp.int32(r), bufs[r], sems[r])

        for s in range(16):
            pltpu.async_copy(
                out_hbm.at[pl.ds(0, 1)], ring.at[pl.ds(s, 1)], osem)

        pltpu.sync_copy(idx_hbm, idx_v)
        pltpu.sync_copy(tail_hbm, tailbuf)

        def collect(g, cur):
            ev = idx_v[pl.ds(g * 16, 16)]
            jv = iota16 + g * 16
            own = ((ev >> 8) & (nw - 1)) == wid
            _compress_store(me_e, cur, ev, own)
            return cur + _compress_store(me_j, cur, jv, own)

        n_me = pl.loop(0, _B // 16, init_carry=jnp.int32(0))(collect)
        n_me_g = (n_me + 15) >> 4

        def chunk_pairs(c):
            def scan(g, cc):
                ev = me_e[pl.ds(g * 16, 16)]
                jv = me_j[pl.ds(g * 16, 16)]
                m = ((ev >> 8) == c) & ((iota16 + g * 16) < n_me)
                packed = ((ev & 255) << 14) | jv
                return cc + _compress_store(cl, cc, packed, m)

            return pl.loop(0, n_me_g, init_carry=jnp.int32(0))(scan)

        def emit_matches(n_pairs, ocnt0, load):
            def one(i, ocnt):
                pk = plsc.load_gather(cl, [jnp.full((16,), i, jnp.int32)])
                colv = pk >> 14
                j = pk[0] & (_B - 1)
                slot = ocnt & 15
                pltpu.make_async_copy(
                    out_hbm.at[pl.ds(0, 1)], ring.at[pl.ds(0, 1)], osem
                ).wait()
                for t in range(_D // 16):
                    ring[slot, 0, pl.ds(t * 16, 16)] = load(t, colv)
                pltpu.async_copy(
                    ring.at[pl.ds(slot, 1)], out_hbm.at[pl.ds(j, 1)], osem)
                return ocnt + 1

            return pl.loop(0, n_pairs, init_carry=ocnt0)(one)

        def process(kk, buf, sem, ocnt):
            pltpu.make_async_copy(
                table_hbm.at[:, pl.ds(0, _CW)], buf, sem).wait()
            c = jnp.where(kk < nmine, wid + kk * nw, -1)
            n_pairs = chunk_pairs(c)

            def load(t, colv):
                rows = iota16 + t * 16
                return plsc.load_gather(buf, [rows, colv])

            return emit_matches(n_pairs, ocnt, load)

        def body(q, ocnt):
            for r in range(3):
                kk = 3 * q + r
                ocnt = process(kk, bufs[r], sems[r], ocnt)
                issue(kk + 3, bufs[r], sems[r])
            return ocnt

        ocnt = pl.loop(0, n_kk // 3, init_carry=jnp.int32(0))(body)
        for r in range(3):
            pltpu.make_async_copy(
                table_hbm.at[:, pl.ds(0, _CW)], bufs[r], sems[r]).wait()

        n_tail = chunk_pairs(jnp.int32(_TAILC))

        def tail_load(t, colv):
            rows = iota16 + t * 16
            return plsc.load_gather(tailbuf, [rows, colv])

        ocnt = emit_matches(n_tail, ocnt, tail_load)

        del ocnt
        for _ in range(16):
            pltpu.make_async_copy(
                out_hbm.at[pl.ds(0, 1)], ring.at[pl.ds(0, 1)], osem).wait()

    return k


def kernel(entities, entity_embeddings):
    info = plsc.get_sparse_core_info()
    fn = _build(info.num_cores, info.num_subcores)
    tail = entity_embeddings[_MAIN:].T
    out = fn(entities.astype(jnp.int32), entity_embeddings.T, tail)
    return out.reshape(_B, _D)
